# Initial kernel scaffold; baseline (speedup 1.0000x reference)
import jax
import jax.numpy as jnp
from jax.experimental import pallas as pl


def kernel(x, edge_index, batch, W1_0, b1_0, W2_0, b2_0, g_0, be_0, W1_1, b1_1, W2_1, b2_1, g_1, be_1, W1_2, b1_2, W2_2, b2_2, g_2, be_2, Wc, bc):
    raise NotImplementedError("write your pallas kernel here")


# speedup vs baseline: 6.2203x; 6.2203x over previous
import functools

import jax
import jax.numpy as jnp
from jax import lax
from jax.experimental import pallas as pl
from jax.experimental.pallas import tpu as pltpu
from jax.experimental.pallas import tpu_sc as plsc

N = 10000
E = 320000
D = 128
G = 128

NW = 32
EPW = E // NW
CH = 128
NFULL = EPW // CH
REM = EPW - NFULL * CH
RPT = 624
TAIL = N - 16 * RPT
ZC = 208


def _sc_scatter(h, src, dst):
    mesh = plsc.VectorSubcoreMesh(core_axis_name="c", subcore_axis_name="s")

    @functools.partial(
        pl.kernel,
        mesh=mesh,
        out_type=(
            jax.ShapeDtypeStruct((N, D), jnp.float32),
            jax.ShapeDtypeStruct((N, D), jnp.float32),
        ),
        scratch_types=[
            pltpu.VMEM((CH,), jnp.int32),
            pltpu.VMEM((CH,), jnp.int32),
            pltpu.VMEM((CH, D), jnp.float32),
            pltpu.VMEM((REM,), jnp.int32),
            pltpu.VMEM((REM,), jnp.int32),
            pltpu.VMEM((REM, D), jnp.float32),
            pltpu.VMEM((ZC, D), jnp.float32),
            pltpu.VMEM_SHARED((N, D), jnp.float32),
            pltpu.SemaphoreType.DMA,
        ],
    )
    def k(h_hbm, src_hbm, dst_hbm, out0, out1,
          si, di, rows, si_r, di_r, rows_r, zv, agg_sh, sem):
        c = lax.axis_index("c")
        s = lax.axis_index("s")
        wid = s * 2 + c

        zrow = jnp.zeros((16,), jnp.float32)

        def zbody(i, carry):
            for cc in range(D // 16):
                zv[i, pl.ds(cc * 16, 16)] = zrow
            return carry

        lax.fori_loop(0, ZC, zbody, 0)
        stripe = s * RPT
        for j in range(RPT // ZC):
            pltpu.sync_copy(zv, agg_sh.at[pl.ds(stripe + j * ZC, ZC)])

        @pl.when(s == 15)
        def _():
            pltpu.sync_copy(zv.at[pl.ds(0, TAIL)],
                            agg_sh.at[pl.ds(16 * RPT, TAIL)])

        plsc.subcore_barrier()

        ebase = wid * EPW

        def chunk(j, carry):
            off = ebase + j * CH
            pltpu.sync_copy(src_hbm.at[pl.ds(off, CH)], si)
            pltpu.sync_copy(dst_hbm.at[pl.ds(off, CH)], di)
            pltpu.async_copy(h_hbm.at[si], rows, sem).wait()
            pltpu.sync_copy(rows, agg_sh.at[di], add=True)
            return carry

        lax.fori_loop(0, NFULL, chunk, 0)
        roff = ebase + NFULL * CH
        pltpu.sync_copy(src_hbm.at[pl.ds(roff, REM)], si_r)
        pltpu.sync_copy(dst_hbm.at[pl.ds(roff, REM)], di_r)
        pltpu.async_copy(h_hbm.at[si_r], rows_r, sem).wait()
        pltpu.sync_copy(rows_r, agg_sh.at[di_r], add=True)
        plsc.subcore_barrier()

        @pl.when(c == 0)
        def _():
            pltpu.sync_copy(agg_sh.at[pl.ds(stripe, RPT)],
                            out0.at[pl.ds(stripe, RPT)])

            @pl.when(s == 15)
            def _():
                pltpu.sync_copy(agg_sh.at[pl.ds(16 * RPT, TAIL)],
                                out0.at[pl.ds(16 * RPT, TAIL)])

        @pl.when(c == 1)
        def _():
            pltpu.sync_copy(agg_sh.at[pl.ds(stripe, RPT)],
                            out1.at[pl.ds(stripe, RPT)])

            @pl.when(s == 15)
            def _():
                pltpu.sync_copy(agg_sh.at[pl.ds(16 * RPT, TAIL)],
                                out1.at[pl.ds(16 * RPT, TAIL)])

    return k(h, src, dst)


R = 1000
GRID = N // R


def _mlp_stats_body(x_r, a0_r, a1_r, w1_r, b1_r, w2_r, b2_r,
                    t_r, sum_r, sq_r, accs, accq):
    i = pl.program_id(0)
    m = x_r[...] + a0_r[...] + a1_r[...]
    hmid = jnp.maximum(
        jnp.dot(m, w1_r[...], preferred_element_type=jnp.float32) + b1_r[...],
        0.0)
    t = jnp.dot(hmid, w2_r[...], preferred_element_type=jnp.float32) + b2_r[...]
    t_r[...] = t
    t3 = t.reshape(R // 8, 8, D)
    ps = jnp.sum(t3, axis=0)
    pq = jnp.sum(t3 * t3, axis=0)

    @pl.when(i == 0)
    def _():
        accs[...] = ps
        accq[...] = pq

    @pl.when(i > 0)
    def _():
        accs[...] += ps
        accq[...] += pq

    @pl.when(i == GRID - 1)
    def _():
        sum_r[...] = accs[...]
        sq_r[...] = accq[...]


def _mlp_stats(x, a0, a1, w1, b1, w2, b2):
    blk = pl.BlockSpec((R, D), lambda i: (i, 0))
    full = pl.BlockSpec((D, D), lambda i: (0, 0))
    vec = pl.BlockSpec((D,), lambda i: (0,))
    return pl.pallas_call(
        _mlp_stats_body,
        grid=(GRID,),
        in_specs=[blk, blk, blk, full, vec, full, vec],
        out_specs=[blk,
                   pl.BlockSpec((8, D), lambda i: (0, 0)),
                   pl.BlockSpec((8, D), lambda i: (0, 0))],
        out_shape=[jax.ShapeDtypeStruct((N, D), jnp.float32),
                   jax.ShapeDtypeStruct((8, D), jnp.float32),
                   jax.ShapeDtypeStruct((8, D), jnp.float32)],
        scratch_shapes=[pltpu.VMEM((8, D), jnp.float32),
                        pltpu.VMEM((8, D), jnp.float32)],
    )(x, a0, a1, w1, b1, w2, b2)


def _bn_relu_body(t_r, sum_r, sq_r, g_r, be_r, h_r):
    mean = jnp.sum(sum_r[...], axis=0) / N
    ex2 = jnp.sum(sq_r[...], axis=0) / N
    var = ex2 - mean * mean
    scale = lax.rsqrt(var + 1e-5) * g_r[...]
    h_r[...] = jnp.maximum((t_r[...] - mean) * scale + be_r[...], 0.0)


def _bn_relu(t, s8, q8, g, be):
    blk = pl.BlockSpec((R, D), lambda i: (i, 0))
    stat = pl.BlockSpec((8, D), lambda i: (0, 0))
    vec = pl.BlockSpec((D,), lambda i: (0,))
    return pl.pallas_call(
        _bn_relu_body,
        grid=(GRID,),
        in_specs=[blk, stat, stat, vec, vec],
        out_specs=blk,
        out_shape=jax.ShapeDtypeStruct((N, D), jnp.float32),
    )(t, s8, q8, g, be)


def _pool_body(h_r, b_r, wc_r, bc_r, out_r, accp, accc):
    i = pl.program_id(0)
    bvec = b_r[0, 0, :]
    gids = lax.broadcasted_iota(jnp.int32, (G, R), 0)
    mask = (gids == bvec[None, :]).astype(jnp.float32)
    ps = jnp.dot(mask, h_r[...], preferred_element_type=jnp.float32)
    pc = jnp.dot(mask, jnp.ones((R, D), jnp.float32),
                 preferred_element_type=jnp.float32)

    @pl.when(i == 0)
    def _():
        accp[...] = ps
        accc[...] = pc

    @pl.when(i > 0)
    def _():
        accp[...] += ps
        accc[...] += pc

    @pl.when(i == GRID - 1)
    def _():
        pooled = accp[...] / jnp.maximum(accc[...], 1.0)
        out_r[...] = (jnp.dot(pooled, wc_r[...],
                              preferred_element_type=jnp.float32) + bc_r[...])


def _pool(h, batch3, wc, bc):
    blk = pl.BlockSpec((R, D), lambda i: (i, 0))
    full = pl.BlockSpec((D, D), lambda i: (0, 0))
    vec = pl.BlockSpec((D,), lambda i: (0,))
    return pl.pallas_call(
        _pool_body,
        grid=(GRID,),
        in_specs=[blk,
                  pl.BlockSpec((1, 1, R), lambda i: (i, 0, 0)),
                  full, vec],
        out_specs=pl.BlockSpec((G, D), lambda i: (0, 0)),
        out_shape=jax.ShapeDtypeStruct((G, D), jnp.float32),
        scratch_shapes=[pltpu.VMEM((G, D), jnp.float32),
                        pltpu.VMEM((G, D), jnp.float32)],
    )(h, batch3, wc, bc)


def kernel(x, edge_index, batch,
           W1_0, b1_0, W2_0, b2_0, g_0, be_0,
           W1_1, b1_1, W2_1, b2_1, g_1, be_1,
           W1_2, b1_2, W2_2, b2_2, g_2, be_2,
           Wc, bc):
    params = [
        (W1_0, b1_0, W2_0, b2_0, g_0, be_0),
        (W1_1, b1_1, W2_1, b2_1, g_1, be_1),
        (W1_2, b1_2, W2_2, b2_2, g_2, be_2),
    ]
    src = edge_index[0]
    dst = edge_index[1]
    h = x
    for (w1, b1, w2, b2, g, be) in params:
        a0, a1 = _sc_scatter(h, src, dst)
        t, s8, q8 = _mlp_stats(h, a0, a1, w1, b1, w2, b2)
        h = _bn_relu(t, s8, q8, g, be)
    batch3 = batch.reshape(GRID, 1, R)
    return _pool(h, batch3, Wc, bc)
